# baseline (speedup 1.0000x reference)
import functools

import jax
import jax.numpy as jnp
from jax import lax
from jax.experimental import pallas as pl
from jax.experimental.pallas import tpu as pltpu
from jax.experimental.pallas import tpu_sc as plsc

N = 10000
NPAD = 10240
E = 160000
EPAD = 163840
NW = 32
EPW = EPAD // NW
CH = 1024
NCH = EPW // CH
BE = 2048
BN = 2048
f32 = jnp.float32


def _sc_gather2_body(src_hbm, tgt_hbm, p_hbm, q_hbm, po_hbm, qo_hbm,
                     idx_s, idx_t, pb, qb, sem1, sem2):
    wid = lax.axis_index("s") * 2 + lax.axis_index("c")

    def body(ch, carry):
        base = pl.multiple_of(wid * EPW + ch * CH, CH)
        pltpu.sync_copy(src_hbm.at[pl.ds(base, CH)], idx_s)
        pltpu.sync_copy(tgt_hbm.at[pl.ds(base, CH)], idx_t)
        c1 = pltpu.async_copy(p_hbm.at[idx_s], pb, sem1)
        c2 = pltpu.async_copy(q_hbm.at[idx_t], qb, sem2)
        c1.wait()
        c2.wait()
        pltpu.sync_copy(pb, po_hbm.at[pl.ds(base, CH)])
        pltpu.sync_copy(qb, qo_hbm.at[pl.ds(base, CH)])
        return carry

    lax.fori_loop(0, NCH, body, 0)


@functools.lru_cache(maxsize=None)
def _sc_mesh():
    return plsc.VectorSubcoreMesh(core_axis_name="c", subcore_axis_name="s")


@functools.lru_cache(maxsize=None)
def _sc_gather2():
    return pl.kernel(
        _sc_gather2_body,
        out_type=[jax.ShapeDtypeStruct((EPAD, 32), f32),
                  jax.ShapeDtypeStruct((EPAD, 32), f32)],
        mesh=_sc_mesh(),
        scratch_types=[pltpu.VMEM((CH,), jnp.int32),
                       pltpu.VMEM((CH,), jnp.int32),
                       pltpu.VMEM((CH, 32), f32),
                       pltpu.VMEM((CH, 32), f32),
                       pltpu.SemaphoreType.DMA,
                       pltpu.SemaphoreType.DMA],
        compiler_params=pltpu.CompilerParams(use_tc_tiling_on_sc=False),
    )


def _sc_gather1_body(src_hbm, t_hbm, out_hbm, idx_s, rb, sem1):
    wid = lax.axis_index("s") * 2 + lax.axis_index("c")

    def body(ch, carry):
        base = pl.multiple_of(wid * EPW + ch * CH, CH)
        pltpu.sync_copy(src_hbm.at[pl.ds(base, CH)], idx_s)
        pltpu.async_copy(t_hbm.at[idx_s], rb, sem1).wait()
        pltpu.sync_copy(rb, out_hbm.at[pl.ds(base, CH)])
        return carry

    lax.fori_loop(0, NCH, body, 0)


@functools.lru_cache(maxsize=None)
def _sc_gather1():
    return pl.kernel(
        _sc_gather1_body,
        out_type=jax.ShapeDtypeStruct((EPAD, 16), f32),
        mesh=_sc_mesh(),
        scratch_types=[pltpu.VMEM((CH,), jnp.int32),
                       pltpu.VMEM((CH, 16), f32),
                       pltpu.SemaphoreType.DMA],
        compiler_params=pltpu.CompilerParams(use_tc_tiling_on_sc=False),
    )


def _sc_scatter_body(src_hbm, tgt_hbm, pos_hbm, zeros_hbm, out_hbm,
                     idx_s, idx_t, vp, acc_t, acc_s):
    cid = lax.axis_index("c")
    sid = lax.axis_index("s")
    wid = sid * 2 + cid

    @pl.when(sid == 0)
    def _():
        pltpu.sync_copy(zeros_hbm, acc_t)
        pltpu.sync_copy(zeros_hbm, acc_s)

    plsc.subcore_barrier()

    def body(ch, carry):
        base = pl.multiple_of(wid * EPW + ch * CH, CH)
        pltpu.sync_copy(tgt_hbm.at[pl.ds(base, CH)], idx_t)
        pltpu.sync_copy(src_hbm.at[pl.ds(base, CH)], idx_s)
        pltpu.sync_copy(pos_hbm.at[pl.ds(base, CH)], vp)
        pltpu.sync_copy(vp, acc_t.at[idx_t], add=True)
        pltpu.sync_copy(vp, acc_s.at[idx_s], add=True)
        return carry

    lax.fori_loop(0, NCH, body, 0)
    plsc.subcore_barrier()
    rows = NPAD // 16
    pltpu.sync_copy(acc_t.at[pl.ds(sid * rows, rows)],
                    out_hbm.at[cid, 0, pl.ds(sid * rows, rows)])
    pltpu.sync_copy(acc_s.at[pl.ds(sid * rows, rows)],
                    out_hbm.at[cid, 1, pl.ds(sid * rows, rows)])


@functools.lru_cache(maxsize=None)
def _sc_scatter():
    return pl.kernel(
        _sc_scatter_body,
        out_type=jax.ShapeDtypeStruct((2, 2, NPAD, 32), f32),
        mesh=_sc_mesh(),
        scratch_types=[pltpu.VMEM((CH,), jnp.int32),
                       pltpu.VMEM((CH,), jnp.int32),
                       pltpu.VMEM((CH, 32), f32),
                       pltpu.VMEM_SHARED((NPAD, 32), f32),
                       pltpu.VMEM_SHARED((NPAD, 32), f32)],
        compiler_params=pltpu.CompilerParams(use_tc_tiling_on_sc=False),
    )


def _pre_body(f_ref, x0_ref, wa_ref, wb_ref, wm_ref, ap_ref,
              pa_ref, qa_ref, mf_ref):
    i = pl.program_id(0)
    f = f_ref[0]
    x0 = x0_ref[:, 0:1]
    s = jnp.where(i == 0, 1.0, 0.0)
    pa_ref[0] = (jnp.dot(f, wa_ref[...], preferred_element_type=f32)
                 + s * x0 * ap_ref[0:1])
    qa_ref[0] = (jnp.dot(f, wb_ref[...], preferred_element_type=f32)
                 + s * x0 * ap_ref[1:2])
    mf_ref[0] = jnp.dot(f, wm_ref[...], preferred_element_type=f32)


def _scal_body(wt_ref, at_ref, sc_ref, s_ref):
    dist = at_ref[0:1]
    de = at_ref[1:2]
    for i in range(4):
        sp = wt_ref[2 * i:2 * i + 1] * sc_ref[2, 0] + sc_ref[0, 0]
        dcn = wt_ref[2 * i + 1:2 * i + 2] * sc_ref[3, 0] + sc_ref[1, 0]
        theta = jnp.abs(de - dcn)
        s_ref[i:i + 1] = jnp.maximum(3.0 * sp * jnp.cos(theta) / dist, 0.0)
    s_ref[4:5] = (dist - sc_ref[4, 0]) / sc_ref[6, 0]
    s_ref[5:6] = (de - sc_ref[5, 0]) / sc_ref[7, 0]
    s_ref[6:7] = jnp.ones_like(dist)
    s_ref[7:8] = jnp.zeros_like(dist)


def _edge_body(po_ref, qo_ref, s_ref, wi_ref, wp_ref, we2_ref, pos_ref):
    z = (po_ref[...] + qo_ref[...]
         + jnp.dot(s_ref[...], wi_ref[...], preferred_element_type=f32,
                   precision=lax.Precision.HIGHEST))
    e1 = jax.nn.sigmoid(z)
    e2 = jax.nn.sigmoid(jnp.dot(e1, we2_ref[...], preferred_element_type=f32)
                        + wp_ref[4:5])
    pos_ref[...] = e2


def _node_body(np_ref, xn_ref, mf_ref, pf_ref, qf_ref,
               wn_ref, wm1_ref, wm2_ref, wm3_ref, wo_ref, bp_ref,
               xo_ref, po_ref, qo_ref):
    sig = jax.nn.sigmoid
    n = (np_ref[0, 0] + np_ref[1, 0]) - (np_ref[0, 1] + np_ref[1, 1])
    g = sig(jnp.dot(n, wn_ref[...], preferred_element_type=f32) + bp_ref[0:1])
    xn = xn_ref[:, 0:1]
    h = sig(jnp.dot(g, wm1_ref[...], preferred_element_type=f32)
            + xn * bp_ref[5:6] + mf_ref[...] + bp_ref[1:2])
    h = sig(jnp.dot(h, wm2_ref[...], preferred_element_type=f32) + bp_ref[2:3])
    h = sig(jnp.dot(h, wm3_ref[...], preferred_element_type=f32) + bp_ref[3:4])
    xnn = jnp.dot(h, wo_ref[...], preferred_element_type=f32) + bp_ref[4:5, 0:8]
    xo_ref[...] = xnn
    x1 = xnn[:, 0:1]
    po_ref[...] = pf_ref[...] + x1 * bp_ref[6:7, 0:32]
    qo_ref[...] = qf_ref[...] + x1 * bp_ref[7:8, 0:32]


def _full(shape):
    return pl.BlockSpec(shape, lambda *a: tuple(0 for _ in shape))


_pre_call = pl.pallas_call(
    _pre_body,
    grid=(4, NPAD // BN),
    in_specs=[
        pl.BlockSpec((1, BN, 128), lambda i, j: (i, j, 0)),
        pl.BlockSpec((BN, 8), lambda i, j: (j, 0)),
        _full((128, 32)),
        _full((128, 32)),
        _full((128, 128)),
        _full((8, 32)),
    ],
    out_specs=[
        pl.BlockSpec((1, BN, 32), lambda i, j: (i, j, 0)),
        pl.BlockSpec((1, BN, 32), lambda i, j: (i, j, 0)),
        pl.BlockSpec((1, BN, 128), lambda i, j: (i, j, 0)),
    ],
    out_shape=[
        jax.ShapeDtypeStruct((4, NPAD, 32), f32),
        jax.ShapeDtypeStruct((4, NPAD, 32), f32),
        jax.ShapeDtypeStruct((4, NPAD, 128), f32),
    ],
)


BS2 = 16384

_scal_call = pl.pallas_call(
    _scal_body,
    grid=(EPAD // BS2,),
    in_specs=[
        pl.BlockSpec((16, BS2), lambda e: (0, e)),
        pl.BlockSpec((8, BS2), lambda e: (0, e)),
        _full((8, 8)),
    ],
    out_specs=pl.BlockSpec((8, BS2), lambda e: (0, e)),
    out_shape=jax.ShapeDtypeStruct((8, EPAD), f32),
)


_edge_call = pl.pallas_call(
    _edge_body,
    grid=(EPAD // BE,),
    in_specs=[
        pl.BlockSpec((BE, 32), lambda e: (e, 0)),
        pl.BlockSpec((BE, 32), lambda e: (e, 0)),
        pl.BlockSpec((BE, 8), lambda e: (e, 0)),
        _full((8, 32)),
        _full((8, 32)),
        _full((32, 32)),
    ],
    out_specs=pl.BlockSpec((BE, 32), lambda e: (e, 0)),
    out_shape=jax.ShapeDtypeStruct((EPAD, 32), f32),
)


_node_call = pl.pallas_call(
    _node_body,
    grid=(NPAD // BN,),
    in_specs=[
        pl.BlockSpec((2, 2, BN, 32), lambda j: (0, 0, j, 0)),
        pl.BlockSpec((BN, 8), lambda j: (j, 0)),
        pl.BlockSpec((BN, 128), lambda j: (j, 0)),
        pl.BlockSpec((BN, 32), lambda j: (j, 0)),
        pl.BlockSpec((BN, 32), lambda j: (j, 0)),
        _full((32, 128)),
        _full((128, 128)),
        _full((128, 128)),
        _full((128, 128)),
        _full((128, 8)),
        _full((8, 128)),
    ],
    out_specs=[
        pl.BlockSpec((BN, 8), lambda j: (j, 0)),
        pl.BlockSpec((BN, 32), lambda j: (j, 0)),
        pl.BlockSpec((BN, 32), lambda j: (j, 0)),
    ],
    out_shape=[
        jax.ShapeDtypeStruct((NPAD, 8), f32),
        jax.ShapeDtypeStruct((NPAD, 32), f32),
        jax.ShapeDtypeStruct((NPAD, 32), f32),
    ],
)


def kernel(pm25_hist, feature, edge_index, edge_attr, wind_mean, wind_std,
           W_e1, b_e1, W_e2, b_e2, W_n, b_n, W_m1, b_m1, W_m2, b_m2,
           W_m3, b_m3, W_o, b_o):
    F4 = feature[0, 8:12]
    Fp = jnp.pad(F4, ((0, 0), (0, NPAD - N), (0, 1)))
    x0p = jnp.pad(pm25_hist[0, -1], ((0, NPAD - N), (0, 7)))
    wt = jnp.transpose(F4[:, :, 125:127], (1, 0, 2)).reshape(N, 8)
    wt = jnp.pad(wt, ((0, NPAD - N), (0, 8)))

    src = jnp.concatenate([edge_index[0],
                           jnp.full((EPAD - E,), N, jnp.int32)])
    tgt = jnp.concatenate([edge_index[1],
                           jnp.full((EPAD - E,), N, jnp.int32)])
    attr_p = jnp.pad(jnp.concatenate(
        [edge_attr, jnp.ones((EPAD - E, 2), f32)], axis=0), ((0, 0), (0, 6)))

    am = jnp.mean(edge_attr, axis=0)
    asd = jnp.std(edge_attr, axis=0, ddof=1)
    sc8 = jnp.broadcast_to(
        jnp.stack([wind_mean[0], wind_mean[1], wind_std[0], wind_std[1],
                   am[0], am[1], asd[0], asd[1]])[:, None], (8, 8))

    wa = jnp.concatenate([W_e1[1:128], jnp.zeros((1, 32), f32)], axis=0)
    wb = jnp.concatenate([W_e1[129:256], jnp.zeros((1, 32), f32)], axis=0)
    wm = jnp.pad(W_m1[14:141], ((0, 1), (0, 64)))
    apack = jnp.pad(jnp.stack([W_e1[0], W_e1[128]]), ((0, 6), (0, 0)))
    wp = jnp.stack([W_e1[256], W_e1[257], W_e1[258],
                    b_e1, jnp.pad(b_e2, (0, 2)),
                    jnp.zeros((32,), f32), jnp.zeros((32,), f32),
                    jnp.zeros((32,), f32)])
    we2 = jnp.pad(W_e2, ((0, 0), (0, 2)))
    z832 = jnp.zeros((8, 32), f32)
    w4 = jnp.stack([
        z832.at[i].set(W_e1[258]).at[4].set(W_e1[256])
            .at[5].set(W_e1[257]).at[6].set(b_e1)
        for i in range(4)])

    wn = jnp.pad(W_n, ((0, 2), (0, 115)))
    wm1 = jnp.pad(W_m1[0:13], ((0, 115), (0, 64)))
    wm2 = jnp.pad(W_m2, ((0, 64), (0, 64)))
    wm3 = jnp.pad(W_m3, ((0, 64), (0, 64)))
    wo = jnp.pad(jnp.tile(W_o, (1, 8)), ((0, 64), (0, 0)))
    bp = jnp.stack([
        jnp.pad(b_n, (0, 115)),
        jnp.pad(b_m1, (0, 64)),
        jnp.pad(b_m2, (0, 64)),
        jnp.pad(b_m3, (0, 64)),
        jnp.full((128,), b_o[0], f32),
        jnp.pad(W_m1[13], (0, 64)),
        jnp.pad(W_e1[0], (0, 96)),
        jnp.pad(W_e1[128], (0, 96)),
    ])
    zeros_acc = jnp.zeros((NPAD, 32), f32)

    pa, qa, mfa = _pre_call(Fp, x0p, wa, wb, wm, apack)
    gwind = _sc_gather1()(src, wt)
    s_t = _scal_call(gwind.T, attr_p.T, sc8)
    s_mat = s_t.T

    xn8 = x0p
    p_tab, q_tab = pa[0], qa[0]
    preds = []
    for i in range(4):
        po, qo = _sc_gather2()(src, tgt, p_tab, q_tab)
        epos = _edge_call(po, qo, s_mat, w4[i], wp, we2)
        nparts = _sc_scatter()(src, tgt, epos, zeros_acc)
        nxt = (i + 1) % 4
        xn8, p_tab, q_tab = _node_call(
            nparts, xn8, mfa[i], pa[nxt], qa[nxt],
            wn, wm1, wm2, wm3, wo, bp)
        preds.append(xn8[:N, 0:1])
    return jnp.stack(preds, axis=0)[None]


# speedup vs baseline: 1.8512x; 1.8512x over previous
import functools

import jax
import jax.numpy as jnp
from jax import lax
from jax.experimental import pallas as pl
from jax.experimental.pallas import tpu as pltpu
from jax.experimental.pallas import tpu_sc as plsc

N = 10000
NPAD = 10240
E = 160000
EPAD = 163840
NW = 32
EPW = EPAD // NW
CH = 1024
NCH = EPW // CH
BE = 2048
BN = 2048
f32 = jnp.float32


def _sc_gather2_body(src_hbm, tgt_hbm, p_hbm, q_hbm, po_hbm, qo_hbm,
                     idx_s, idx_t, pb, qb, sem1, sem2):
    wid = lax.axis_index("s") * 2 + lax.axis_index("c")

    def body(ch, carry):
        base = pl.multiple_of(wid * EPW + ch * CH, CH)
        pltpu.sync_copy(src_hbm.at[pl.ds(base, CH)], idx_s)
        pltpu.sync_copy(tgt_hbm.at[pl.ds(base, CH)], idx_t)
        c1 = pltpu.async_copy(p_hbm.at[idx_s], pb, sem1)
        c2 = pltpu.async_copy(q_hbm.at[idx_t], qb, sem2)
        c1.wait()
        c2.wait()
        pltpu.sync_copy(pb, po_hbm.at[pl.ds(base, CH)])
        pltpu.sync_copy(qb, qo_hbm.at[pl.ds(base, CH)])
        return carry

    lax.fori_loop(0, NCH, body, 0)


@functools.lru_cache(maxsize=None)
def _sc_mesh():
    return plsc.VectorSubcoreMesh(core_axis_name="c", subcore_axis_name="s")


@functools.lru_cache(maxsize=None)
def _sc_gather2():
    return pl.kernel(
        _sc_gather2_body,
        out_type=[jax.ShapeDtypeStruct((EPAD, 32), f32),
                  jax.ShapeDtypeStruct((EPAD, 32), f32)],
        mesh=_sc_mesh(),
        scratch_types=[pltpu.VMEM((CH,), jnp.int32),
                       pltpu.VMEM((CH,), jnp.int32),
                       pltpu.VMEM((CH, 32), f32),
                       pltpu.VMEM((CH, 32), f32),
                       pltpu.SemaphoreType.DMA,
                       pltpu.SemaphoreType.DMA],
        compiler_params=pltpu.CompilerParams(use_tc_tiling_on_sc=False),
    )


def _sc_gather1_body(src_hbm, t_hbm, out_hbm, idx_s, rb, sem1):
    wid = lax.axis_index("s") * 2 + lax.axis_index("c")

    def body(ch, carry):
        base = pl.multiple_of(wid * EPW + ch * CH, CH)
        pltpu.sync_copy(src_hbm.at[pl.ds(base, CH)], idx_s)
        pltpu.async_copy(t_hbm.at[idx_s], rb, sem1).wait()
        pltpu.sync_copy(rb, out_hbm.at[pl.ds(base, CH)])
        return carry

    lax.fori_loop(0, NCH, body, 0)


@functools.lru_cache(maxsize=None)
def _sc_gather1():
    return pl.kernel(
        _sc_gather1_body,
        out_type=jax.ShapeDtypeStruct((EPAD, 16), f32),
        mesh=_sc_mesh(),
        scratch_types=[pltpu.VMEM((CH,), jnp.int32),
                       pltpu.VMEM((CH, 16), f32),
                       pltpu.SemaphoreType.DMA],
        compiler_params=pltpu.CompilerParams(use_tc_tiling_on_sc=False),
    )


def _sc_scatter_body(src_hbm, tgt_hbm, pos_hbm, zeros_hbm, out_hbm,
                     idx_s, idx_t, vp, acc_t, acc_s):
    cid = lax.axis_index("c")
    sid = lax.axis_index("s")
    wid = sid * 2 + cid

    @pl.when(sid == 0)
    def _():
        pltpu.sync_copy(zeros_hbm, acc_t)
        pltpu.sync_copy(zeros_hbm, acc_s)

    plsc.subcore_barrier()

    def body(ch, carry):
        base = pl.multiple_of(wid * EPW + ch * CH, CH)
        pltpu.sync_copy(tgt_hbm.at[pl.ds(base, CH)], idx_t)
        pltpu.sync_copy(src_hbm.at[pl.ds(base, CH)], idx_s)
        pltpu.sync_copy(pos_hbm.at[pl.ds(base, CH)], vp)
        pltpu.sync_copy(vp, acc_t.at[idx_t], add=True)
        pltpu.sync_copy(vp, acc_s.at[idx_s], add=True)
        return carry

    lax.fori_loop(0, NCH, body, 0)
    plsc.subcore_barrier()
    rows = NPAD // 16
    pltpu.sync_copy(acc_t.at[pl.ds(sid * rows, rows)],
                    out_hbm.at[cid, 0, pl.ds(sid * rows, rows)])
    pltpu.sync_copy(acc_s.at[pl.ds(sid * rows, rows)],
                    out_hbm.at[cid, 1, pl.ds(sid * rows, rows)])


@functools.lru_cache(maxsize=None)
def _sc_scatter():
    return pl.kernel(
        _sc_scatter_body,
        out_type=jax.ShapeDtypeStruct((2, 2, NPAD, 32), f32),
        mesh=_sc_mesh(),
        scratch_types=[pltpu.VMEM((CH,), jnp.int32),
                       pltpu.VMEM((CH,), jnp.int32),
                       pltpu.VMEM((CH, 32), f32),
                       pltpu.VMEM_SHARED((NPAD, 32), f32),
                       pltpu.VMEM_SHARED((NPAD, 32), f32)],
        compiler_params=pltpu.CompilerParams(use_tc_tiling_on_sc=False),
    )


def _pre_body(f_ref, x0_ref, wa_ref, wb_ref, wm_ref, ap_ref,
              pa_ref, qa_ref, mf_ref):
    i = pl.program_id(0)
    f = f_ref[0]
    x0 = x0_ref[:, 0:1]
    s = jnp.where(i == 0, 1.0, 0.0)
    pa_ref[0] = (jnp.dot(f, wa_ref[...], preferred_element_type=f32)
                 + s * x0 * ap_ref[0:1])
    qa_ref[0] = (jnp.dot(f, wb_ref[...], preferred_element_type=f32)
                 + s * x0 * ap_ref[1:2])
    mf_ref[0] = jnp.dot(f, wm_ref[...], preferred_element_type=f32)


def _scal_body(wt_ref, at_ref, sc_ref, s_ref):
    dist = at_ref[0:1]
    de = at_ref[1:2]
    for i in range(4):
        sp = wt_ref[2 * i:2 * i + 1] * sc_ref[2, 0] + sc_ref[0, 0]
        dcn = wt_ref[2 * i + 1:2 * i + 2] * sc_ref[3, 0] + sc_ref[1, 0]
        theta = jnp.abs(de - dcn)
        s_ref[i:i + 1] = jnp.maximum(3.0 * sp * jnp.cos(theta) / dist, 0.0)
    s_ref[4:5] = (dist - sc_ref[4, 0]) / sc_ref[6, 0]
    s_ref[5:6] = (de - sc_ref[5, 0]) / sc_ref[7, 0]
    s_ref[6:7] = jnp.ones_like(dist)
    s_ref[7:8] = jnp.zeros_like(dist)


def _edge_body(po_ref, qo_ref, s_ref, wi_ref, wp_ref, we2_ref, pos_ref):
    z = (po_ref[...] + qo_ref[...]
         + jnp.dot(s_ref[...], wi_ref[...], preferred_element_type=f32,
                   precision=lax.Precision.HIGHEST))
    e1 = jax.nn.sigmoid(z)
    e2 = jax.nn.sigmoid(jnp.dot(e1, we2_ref[...], preferred_element_type=f32)
                        + wp_ref[4:5])
    pos_ref[...] = e2


EPAD4 = EPAD // 4
BE4 = 2048


def _node_body(np_ref, xn_ref, mf_ref, pf_ref, qf_ref,
               wn_ref, wm1_ref, wm2_ref, wm3_ref, wo_ref, bp_ref,
               xo_ref, po_ref, qo_ref):
    sig = jax.nn.sigmoid
    n = (np_ref[0, 0] + np_ref[1, 0]) - (np_ref[0, 1] + np_ref[1, 1])
    g = sig(jnp.dot(n, wn_ref[...], preferred_element_type=f32) + bp_ref[0:1])
    xn = xn_ref[:, 0:1]
    h = sig(jnp.dot(g, wm1_ref[...], preferred_element_type=f32)
            + xn * bp_ref[5:6] + mf_ref[...] + bp_ref[1:2])
    h = sig(jnp.dot(h, wm2_ref[...], preferred_element_type=f32) + bp_ref[2:3])
    h = sig(jnp.dot(h, wm3_ref[...], preferred_element_type=f32) + bp_ref[3:4])
    xnn = jnp.dot(h, wo_ref[...], preferred_element_type=f32) + bp_ref[4:5, 0:8]
    xo_ref[...] = xnn
    x1 = xnn[:, 0:1]
    po_ref[...] = pf_ref[...] + x1 * bp_ref[6:7, 0:32]
    qo_ref[...] = qf_ref[...] + x1 * bp_ref[7:8, 0:32]


def _full(shape):
    return pl.BlockSpec(shape, lambda *a: tuple(0 for _ in shape))


_pre_call = pl.pallas_call(
    _pre_body,
    grid=(4, NPAD // BN),
    in_specs=[
        pl.BlockSpec((1, BN, 128), lambda i, j: (i, j, 0)),
        pl.BlockSpec((BN, 8), lambda i, j: (j, 0)),
        _full((128, 32)),
        _full((128, 32)),
        _full((128, 128)),
        _full((8, 32)),
    ],
    out_specs=[
        pl.BlockSpec((1, BN, 32), lambda i, j: (i, j, 0)),
        pl.BlockSpec((1, BN, 32), lambda i, j: (i, j, 0)),
        pl.BlockSpec((1, BN, 128), lambda i, j: (i, j, 0)),
    ],
    out_shape=[
        jax.ShapeDtypeStruct((4, NPAD, 32), f32),
        jax.ShapeDtypeStruct((4, NPAD, 32), f32),
        jax.ShapeDtypeStruct((4, NPAD, 128), f32),
    ],
)


BS2 = 16384

_scal_call = pl.pallas_call(
    _scal_body,
    grid=(EPAD // BS2,),
    in_specs=[
        pl.BlockSpec((16, BS2), lambda e: (0, e)),
        pl.BlockSpec((8, BS2), lambda e: (0, e)),
        _full((8, 8)),
    ],
    out_specs=pl.BlockSpec((8, BS2), lambda e: (0, e)),
    out_shape=jax.ShapeDtypeStruct((8, EPAD), f32),
)


_edge_call = pl.pallas_call(
    _edge_body,
    grid=(EPAD4 // BE4,),
    in_specs=[
        pl.BlockSpec((BE4, 128), lambda e: (e, 0)),
        pl.BlockSpec((BE4, 128), lambda e: (e, 0)),
        pl.BlockSpec((BE4, 32), lambda e: (e, 0)),
        _full((32, 128)),
        _full((8, 128)),
        _full((128, 128)),
    ],
    out_specs=pl.BlockSpec((BE4, 128), lambda e: (e, 0)),
    out_shape=jax.ShapeDtypeStruct((EPAD4, 128), f32),
)


_node_call = pl.pallas_call(
    _node_body,
    grid=(NPAD // BN,),
    in_specs=[
        pl.BlockSpec((2, 2, BN, 32), lambda j: (0, 0, j, 0)),
        pl.BlockSpec((BN, 8), lambda j: (j, 0)),
        pl.BlockSpec((BN, 128), lambda j: (j, 0)),
        pl.BlockSpec((BN, 32), lambda j: (j, 0)),
        pl.BlockSpec((BN, 32), lambda j: (j, 0)),
        _full((32, 128)),
        _full((128, 128)),
        _full((128, 128)),
        _full((128, 128)),
        _full((128, 8)),
        _full((8, 128)),
    ],
    out_specs=[
        pl.BlockSpec((BN, 8), lambda j: (j, 0)),
        pl.BlockSpec((BN, 32), lambda j: (j, 0)),
        pl.BlockSpec((BN, 32), lambda j: (j, 0)),
    ],
    out_shape=[
        jax.ShapeDtypeStruct((NPAD, 8), f32),
        jax.ShapeDtypeStruct((NPAD, 32), f32),
        jax.ShapeDtypeStruct((NPAD, 32), f32),
    ],
)


def kernel(pm25_hist, feature, edge_index, edge_attr, wind_mean, wind_std,
           W_e1, b_e1, W_e2, b_e2, W_n, b_n, W_m1, b_m1, W_m2, b_m2,
           W_m3, b_m3, W_o, b_o):
    F4 = feature[0, 8:12]
    Fp = jnp.pad(F4, ((0, 0), (0, NPAD - N), (0, 1)))
    x0p = jnp.pad(pm25_hist[0, -1], ((0, NPAD - N), (0, 7)))
    wt = jnp.transpose(F4[:, :, 125:127], (1, 0, 2)).reshape(N, 8)
    wt = jnp.pad(wt, ((0, NPAD - N), (0, 8)))

    src = jnp.concatenate([edge_index[0],
                           jnp.full((EPAD - E,), N, jnp.int32)])
    tgt = jnp.concatenate([edge_index[1],
                           jnp.full((EPAD - E,), N, jnp.int32)])
    attr_p = jnp.pad(jnp.concatenate(
        [edge_attr, jnp.ones((EPAD - E, 2), f32)], axis=0), ((0, 0), (0, 6)))

    am = jnp.mean(edge_attr, axis=0)
    asd = jnp.std(edge_attr, axis=0, ddof=1)
    sc8 = jnp.broadcast_to(
        jnp.stack([wind_mean[0], wind_mean[1], wind_std[0], wind_std[1],
                   am[0], am[1], asd[0], asd[1]])[:, None], (8, 8))

    wa = jnp.concatenate([W_e1[1:128], jnp.zeros((1, 32), f32)], axis=0)
    wb = jnp.concatenate([W_e1[129:256], jnp.zeros((1, 32), f32)], axis=0)
    wm = jnp.pad(W_m1[14:141], ((0, 1), (0, 64)))
    apack = jnp.pad(jnp.stack([W_e1[0], W_e1[128]]), ((0, 6), (0, 0)))
    wp = jnp.stack([W_e1[256], W_e1[257], W_e1[258],
                    b_e1, jnp.pad(b_e2, (0, 2)),
                    jnp.zeros((32,), f32), jnp.zeros((32,), f32),
                    jnp.zeros((32,), f32)])
    we2 = jnp.pad(W_e2, ((0, 0), (0, 2)))
    z832 = jnp.zeros((8, 32), f32)
    eye4 = jnp.eye(4, dtype=f32)
    w4 = jnp.stack([
        jnp.kron(eye4, z832.at[i].set(W_e1[258]).at[4].set(W_e1[256])
                 .at[5].set(W_e1[257]).at[6].set(b_e1))
        for i in range(4)])
    we2bd = jnp.kron(eye4, we2)
    wp4 = jnp.tile(wp, (1, 4))

    wn = jnp.pad(W_n, ((0, 2), (0, 115)))
    wm1 = jnp.pad(W_m1[0:13], ((0, 115), (0, 64)))
    wm2 = jnp.pad(W_m2, ((0, 64), (0, 64)))
    wm3 = jnp.pad(W_m3, ((0, 64), (0, 64)))
    wo = jnp.pad(jnp.tile(W_o, (1, 8)), ((0, 64), (0, 0)))
    bp = jnp.stack([
        jnp.pad(b_n, (0, 115)),
        jnp.pad(b_m1, (0, 64)),
        jnp.pad(b_m2, (0, 64)),
        jnp.pad(b_m3, (0, 64)),
        jnp.full((128,), b_o[0], f32),
        jnp.pad(W_m1[13], (0, 64)),
        jnp.pad(W_e1[0], (0, 96)),
        jnp.pad(W_e1[128], (0, 96)),
    ])
    zeros_acc = jnp.zeros((NPAD, 32), f32)

    pa, qa, mfa = _pre_call(Fp, x0p, wa, wb, wm, apack)
    gwind = _sc_gather1()(src, wt)
    s_t = _scal_call(gwind.T, attr_p.T, sc8)
    s4 = s_t.T.reshape(EPAD4, 32)

    xn8 = x0p
    p_tab, q_tab = pa[0], qa[0]
    preds = []
    for i in range(4):
        po, qo = _sc_gather2()(src, tgt, p_tab, q_tab)
        epos = _edge_call(po.reshape(EPAD4, 128), qo.reshape(EPAD4, 128),
                          s4, w4[i], wp4, we2bd)
        nparts = _sc_scatter()(src, tgt, epos.reshape(EPAD, 32), zeros_acc)
        nxt = (i + 1) % 4
        xn8, p_tab, q_tab = _node_call(
            nparts, xn8, mfa[i], pa[nxt], qa[nxt],
            wn, wm1, wm2, wm3, wo, bp)
        preds.append(xn8[:N, 0:1])
    return jnp.stack(preds, axis=0)[None]
